# baseline (speedup 1.0000x reference)
import functools

import jax
import jax.numpy as jnp
from jax import lax
from jax.experimental import pallas as pl
from jax.experimental.pallas import tpu as pltpu
from jax.experimental.pallas import tpu_sc as plsc

_EPS = 1e-8


def _sqrt_nt(y, s, iters):
    for _ in range(iters):
        s = 0.5 * (s + y / s)
    return s


def _rsqrt_nt(y, s, iters):
    for _ in range(iters):
        s = s * (1.5 - 0.5 * y * s * s)
    return s


def _sc_params_body(nb, gamma_hbm, ld_hbm, idx_hbm, mean_hbm, var_hbm,
                    r_out, off_out,
                    idx_v, g_v, ld_v, mv_v, r_v, off_v, sem, sem2):
    cid = lax.axis_index("c")
    sid = lax.axis_index("s")
    cp = g_v.shape[1]
    nc = cp // 16

    @pl.when(jnp.logical_and(cid == 0, sid == 0))
    def _():
        pltpu.sync_copy(idx_hbm, idx_v)
        cp_g = pltpu.async_copy(gamma_hbm.at[idx_v], g_v, sem)
        cp_l = pltpu.async_copy(ld_hbm.at[idx_v], ld_v, sem2)
        pltpu.sync_copy(mean_hbm, mv_v.at[0])
        pltpu.sync_copy(var_hbm, mv_v.at[1])
        cp_g.wait()
        cp_l.wait()
        for c in range(nc):
            sl = pl.ds(c * 16, 16)
            gt = g_v[nb, sl]
            ldt = ld_v[nb, sl]
            m = mv_v[0, sl]
            var_ = mv_v[1, sl]
            yv = var_ + _EPS
            sv = _sqrt_nt(yv, 0.5 * (1.0 + yv), 10)
            rt_half = jnp.exp(0.5 * ldt)
            for b in range(nb):
                gb = g_v[b, sl]
                ldb = ld_v[b, sl]
                db = jnp.exp(ldb)
                seed = jnp.exp(-0.5 * ldb)
                r = rt_half * _rsqrt_nt(db + _EPS, seed, 3)
                off = m * (1.0 - r) + sv * (gt - gb * r)
                dst = pl.ds(b * cp + c * 16, 16)
                r_v[dst] = r
                off_v[dst] = off
        cp_r = pltpu.async_copy(r_v, r_out, sem)
        cp_o = pltpu.async_copy(off_v, off_out, sem2)
        cp_r.wait()
        cp_o.wait()


def _sc_params(gamma_p, ld_p, idx, mean_p, var_p, nb):
    n = idx.shape[0]
    cp = gamma_p.shape[1]
    fn = pl.kernel(
        functools.partial(_sc_params_body, nb),
        mesh=plsc.VectorSubcoreMesh(core_axis_name="c", subcore_axis_name="s"),
        out_type=[jax.ShapeDtypeStruct((nb * cp,), jnp.float32),
                  jax.ShapeDtypeStruct((nb * cp,), jnp.float32)],
        scratch_types=[pltpu.VMEM((n,), jnp.int32),
                       pltpu.VMEM((n, cp), jnp.float32),
                       pltpu.VMEM((n, cp), jnp.float32),
                       pltpu.VMEM((2, cp), jnp.float32),
                       pltpu.VMEM((nb * cp,), jnp.float32),
                       pltpu.VMEM((nb * cp,), jnp.float32),
                       pltpu.SemaphoreType.DMA,
                       pltpu.SemaphoreType.DMA],
    )
    return fn(gamma_p, ld_p, idx, mean_p, var_p)


def _apply_body(c, cp, r_sm, off_sm, x_ref, o_ref):
    i = pl.program_id(0)
    rb = x_ref.shape[0]
    for j in range(rb):
        k = i * rb + j
        slot = (k // c) * cp + (k % c)
        o_ref[j] = x_ref[j] * r_sm[slot] + off_sm[slot]


def _apply(x3, r_row, off_row, c, cp, rb):
    n, h, w = x3.shape
    sm_spec = pl.BlockSpec(memory_space=pltpu.SMEM)
    return pl.pallas_call(
        functools.partial(_apply_body, c, cp),
        grid=(n // rb,),
        in_specs=[
            sm_spec, sm_spec,
            pl.BlockSpec((rb, h, w), lambda i: (i, 0, 0)),
        ],
        out_specs=pl.BlockSpec((rb, h, w), lambda i: (i, 0, 0)),
        out_shape=jax.ShapeDtypeStruct((n, h, w), jnp.float32),
        compiler_params=pltpu.CompilerParams(
            dimension_semantics=("parallel",)),
    )(r_row, off_row, x3)


def kernel(x, batch, gamma, log_delta, running_mean, running_var, target_batch):
    b, c, h, w = x.shape

    tgt = jnp.full((8,), target_batch, dtype=jnp.int32)
    idx = jnp.concatenate([batch.astype(jnp.int32), tgt])

    cp = ((c + 127) // 128) * 128
    pad = ((0, 0), (0, cp - c))
    padv = ((0, cp - c),)
    r_row, off_row = _sc_params(
        jnp.pad(gamma, pad), jnp.pad(log_delta, pad), idx,
        jnp.pad(running_mean, padv), jnp.pad(running_var, padv),
        nb=b)

    x3 = x.reshape(b * c, h, w)
    out = _apply(x3, r_row, off_row, c, cp, rb=64)
    return out.reshape(b, c, h, w)
